# baseline (speedup 1.0000x reference)
import jax
import jax.numpy as jnp
from jax import lax
from jax.experimental import pallas as pl
from jax.experimental.pallas import tpu as pltpu
from jax.experimental.pallas import tpu_sc as plsc

_N = 10000
_E = 320000
_D_IN = 128
_HID = 16
_HEADS = 8
_D1 = _HEADS * _HID
_D_OUT = 64

_NCORES = 2
_NSUB = 16
_NPAD = 10240
_ROWS_PER_TILE = _NPAD // _NSUB
_TRASH = _NPAD - 8

_B = 32
_H = 16
_EINT_LEN = 641024

_CHUNKS1 = (_E // _NSUB) // _B
_CHUNKS2 = 313


def _leaky(v):
    return jnp.maximum(v, 0.2 * v)


def _edge_pass(t_hbm, eint_hbm, z_hbm, acc_out, sets, acc_sh,
               base, chunks, gsrc, gdst, n_mul, w_col, n_wlanes):
    sid = lax.axis_index("s")
    cid = lax.axis_index("c")
    r0 = sid * _ROWS_PER_TILE
    pltpu.sync_copy(z_hbm.at[pl.ds(r0, _ROWS_PER_TILE), :],
                    acc_sh.at[pl.ds(r0, _ROWS_PER_TILE), :])
    plsc.subcore_barrier()

    lane = lax.iota(jnp.int32, 16)
    gsrc_v = jnp.full((16,), gsrc, jnp.int32)
    gdst_v = jnp.full((16,), gdst, jnp.int32)

    def _fire_i(c, st):
        off = 2 * (base + c * _B)
        pltpu.async_copy(eint_hbm.at[pl.ds(off, 2 * _B)], st[0], st[5])

    def _wait_i(c, st):
        off = 2 * (base + c * _B)
        pltpu.make_async_copy(eint_hbm.at[pl.ds(off, 2 * _B)], st[0], st[5]).wait()

    def _stage(c, st):
        idxland, idxg, dsts_a, dsts_b = st[0], st[1], st[2], st[3]
        idxg[pl.ds(0, 16)] = idxland[pl.ds(0, 16)] + gsrc_v
        idxg[pl.ds(16, 16)] = idxland[pl.ds(16, 16)] + gsrc_v
        d0 = idxland[pl.ds(32, 16)]
        d1 = idxland[pl.ds(48, 16)]
        idxg[pl.ds(32, 16)] = d0 + gdst_v
        idxg[pl.ds(48, 16)] = d1 + gdst_v
        pos0 = jnp.full((16,), base + c * _B, jnp.int32) + lane
        dsts_a[...] = jnp.where(pos0 < _E, d0, _TRASH)
        dsts_b[...] = jnp.where(pos0 + 16 < _E, d1, _TRASH)

    def _fire_g(st):
        pltpu.async_copy(t_hbm.at[st[1]], st[4], st[6])

    def _wait_g(st):
        pltpu.make_async_copy(t_hbm.at[st[1]], st[4], st[6]).wait()

    def _compute_fire_s(st):
        rows = st[4]

        def _edge(k, cc):
            w16 = jnp.exp(_leaky(rows[k, pl.ds(64, 16)] + rows[32 + k, pl.ds(0, 16)]))
            rows[k, pl.ds(w_col, 16)] = jnp.where(lane < n_wlanes, w16, 0.0)
            for h in range(n_mul):
                wv = jnp.full((16,), w16[h if n_wlanes > 1 else 0], jnp.float32)
                rows[k, pl.ds(h * 16, 16)] = rows[k, pl.ds(h * 16, 16)] * wv
            return cc

        lax.fori_loop(0, _H, _edge, 0, unroll=8)
        pltpu.async_copy(rows.at[pl.ds(0, _H)], acc_sh.at[st[2]], st[7], add=True)
        lax.fori_loop(_H, _B, _edge, 0, unroll=8)
        pltpu.async_copy(rows.at[pl.ds(_H, _H)], acc_sh.at[st[3]], st[7], add=True)

    def _wait_s(st):
        pltpu.make_async_copy(st[4].at[pl.ds(0, _H)], acc_sh.at[st[2]], st[7]).wait()
        pltpu.make_async_copy(st[4].at[pl.ds(_H, _H)], acc_sh.at[st[3]], st[7]).wait()

    def _phase(c, st, st2):
        _wait_g(st)

        @pl.when(c + 2 < chunks)
        def _():
            @pl.when(c >= 2)
            def _():
                _wait_s(st2)

            _wait_i(c + 2, st2)
            _stage(c + 2, st2)
            _fire_g(st2)

        @pl.when(c + 4 < chunks)
        def _():
            _fire_i(c + 4, st)

        _compute_fire_s(st)

    for k in range(4):
        _fire_i(k, sets[k])
    for k in range(2):
        _wait_i(k, sets[k])
        _stage(k, sets[k])
        _fire_g(sets[k])

    def _quad(j, carry):
        c0 = 4 * j
        _phase(c0, sets[0], sets[2])
        _phase(c0 + 1, sets[1], sets[3])
        _phase(c0 + 2, sets[2], sets[0])
        _phase(c0 + 3, sets[3], sets[1])
        return carry

    q = chunks // 4
    lax.fori_loop(0, q, _quad, 0)
    for r in range(4 * q, chunks):
        _phase(r, sets[r % 4], sets[(r + 2) % 4])
    for k in range(4):
        _wait_s(sets[k])

    plsc.subcore_barrier()
    pltpu.sync_copy(acc_sh.at[pl.ds(r0, _ROWS_PER_TILE), :],
                    acc_out.at[cid, pl.ds(r0, _ROWS_PER_TILE), :])


def _sets(args):
    return [tuple(args[5 * k:5 * k + 5]) + tuple(args[20 + 3 * k:23 + 3 * k])
            for k in range(4)]


def _sc_edges1(t_hbm, eint_hbm, z_hbm, acc_out,
               a0, a1, a2, a3, a4, b0, b1, b2, b3, b4,
               c0, c1, c2, c3, c4, d0, d1, d2, d3, d4,
               acc_sh, mi0, mg0, ms0, mi1, mg1, ms1,
               mi2, mg2, ms2, mi3, mg3, ms3):
    cid = lax.axis_index("c")
    sid = lax.axis_index("s")
    scr = [a0, a1, a2, a3, a4, b0, b1, b2, b3, b4,
           c0, c1, c2, c3, c4, d0, d1, d2, d3, d4,
           mi0, mg0, ms0, mi1, mg1, ms1, mi2, mg2, ms2, mi3, mg3, ms3]
    _edge_pass(t_hbm, eint_hbm, z_hbm, acc_out, _sets(scr), acc_sh,
               sid * (_E // _NSUB), _CHUNKS1,
               gsrc=cid * _N, gdst=2 * _N + cid * _N,
               n_mul=4, w_col=64, n_wlanes=4)


def _sc_edges2(t_hbm, eint_hbm, z_hbm, acc_out,
               a0, a1, a2, a3, a4, b0, b1, b2, b3, b4,
               c0, c1, c2, c3, c4, d0, d1, d2, d3, d4,
               acc_sh, mi0, mg0, ms0, mi1, mg1, ms1,
               mi2, mg2, ms2, mi3, mg3, ms3):
    cid = lax.axis_index("c")
    sid = lax.axis_index("s")
    scr = [a0, a1, a2, a3, a4, b0, b1, b2, b3, b4,
           c0, c1, c2, c3, c4, d0, d1, d2, d3, d4,
           mi0, mg0, ms0, mi1, mg1, ms1, mi2, mg2, ms2, mi3, mg3, ms3]
    wid = cid * _NSUB + sid
    _edge_pass(t_hbm, eint_hbm, z_hbm, acc_out, _sets(scr), acc_sh,
               wid * (_CHUNKS2 * _B), _CHUNKS2,
               gsrc=0, gdst=_N,
               n_mul=4, w_col=64, n_wlanes=1)


def _make_sc_call(body):
    mesh = plsc.VectorSubcoreMesh(core_axis_name="c", subcore_axis_name="s")
    buf = []
    for _ in range(4):
        buf += [
            pltpu.VMEM((2 * _B,), jnp.int32),
            pltpu.VMEM((2 * _B,), jnp.int32),
            pltpu.VMEM((_H,), jnp.int32),
            pltpu.VMEM((_H,), jnp.int32),
            pltpu.VMEM((2 * _B, 128), jnp.float32),
        ]
    return pl.kernel(
        body,
        out_type=jax.ShapeDtypeStruct((_NCORES, _NPAD, 128), jnp.float32),
        mesh=mesh,
        scratch_types=[
            *buf,
            pltpu.VMEM_SHARED((_NPAD, 128), jnp.float32),
            *([pltpu.SemaphoreType.DMA] * 12),
        ],
    )


def _tc_stage_a(x_ref, w1_ref, aproj_ref, t_out):
    h = jnp.dot(x_ref[...], w1_ref[...], preferred_element_type=jnp.float32)
    att = jnp.dot(h, aproj_ref[...], preferred_element_type=jnp.float32)
    n = h.shape[0]
    z56 = jnp.zeros((n, 56), jnp.float32)
    z124 = jnp.zeros((n, 124), jnp.float32)
    t_out[0:10000, :] = jnp.concatenate(
        [h[:, 0:64], att[:, 0:4], att[:, 8:12], z56], axis=1)
    t_out[10000:20000, :] = jnp.concatenate(
        [h[:, 64:128], att[:, 4:8], att[:, 12:16], z56], axis=1)
    t_out[20000:30000, :] = jnp.concatenate([att[:, 8:12], z124], axis=1)
    t_out[30000:40000, :] = jnp.concatenate([att[:, 12:16], z124], axis=1)


def _tc_stage_b(acc_ref, t1_ref, b1_ref, w2_ref, r_ref, a2_ref, t_out):
    h1 = jnp.concatenate([t1_ref[0:10000, 0:64], t1_ref[10000:20000, 0:64]], axis=1)
    as1 = jnp.concatenate([t1_ref[0:10000, 64:68], t1_ref[10000:20000, 64:68]], axis=1)
    ad1 = jnp.concatenate([t1_ref[0:10000, 68:72], t1_ref[10000:20000, 68:72]], axis=1)
    wself = jnp.exp(_leaky(as1 + ad1))
    num = jnp.concatenate([acc_ref[0, 0:10000, 0:64], acc_ref[1, 0:10000, 0:64]], axis=1)
    den = jnp.concatenate([acc_ref[0, 0:10000, 64:68], acc_ref[1, 0:10000, 64:68]], axis=1)
    den = den + wself
    wexp = jnp.dot(wself, r_ref[...], preferred_element_type=jnp.float32)
    dexp = jnp.dot(den, r_ref[...], preferred_element_type=jnp.float32)
    num = num + h1 * wexp
    z = num / dexp + b1_ref[...]
    z = jnp.where(z > 0, z, jnp.exp(jnp.minimum(z, 0.0)) - 1.0)
    h2 = jnp.dot(z, w2_ref[...], preferred_element_type=jnp.float32)
    att2 = jnp.dot(h2, a2_ref[...], preferred_element_type=jnp.float32)
    n = h2.shape[0]
    t_out[0:10000, :] = jnp.concatenate(
        [h2, att2, jnp.zeros((n, 62), jnp.float32)], axis=1)
    t_out[10000:20000, :] = jnp.concatenate(
        [att2[:, 1:2], jnp.zeros((n, 127), jnp.float32)], axis=1)


def _tc_stage_c(acc_ref, t2_ref, b2_ref, out_ref):
    h2 = t2_ref[0:10000, 0:64]
    wself = jnp.exp(_leaky(t2_ref[0:10000, 64:65] + t2_ref[0:10000, 65:66]))
    num = acc_ref[0, 0:10000, 0:64] + acc_ref[1, 0:10000, 0:64] + h2 * wself
    den = acc_ref[0, 0:10000, 64:65] + acc_ref[1, 0:10000, 64:65] + wself
    o = num / den + b2_ref[...]
    m = jnp.max(o, axis=1, keepdims=True)
    lse = jnp.log(jnp.sum(jnp.exp(o - m), axis=1, keepdims=True)) + m
    out_ref[...] = o - lse


def kernel(x, edge_index, W1, att_src1, att_dst1, b1, W2, att_src2, att_dst2, b2):
    f32 = jnp.float32
    eye_h = jnp.eye(_HEADS, dtype=f32)
    t_src = (eye_h[:, None, :] * att_src1.astype(f32).T[None, :, :]).reshape(_D1, _HEADS)
    t_dst = (eye_h[:, None, :] * att_dst1.astype(f32).T[None, :, :]).reshape(_D1, _HEADS)
    aproj = jnp.concatenate([t_src, t_dst], axis=1)
    r_mat = jnp.kron(jnp.eye(_HEADS, dtype=f32), jnp.ones((1, _HID), f32))
    a2 = jnp.concatenate([att_src2.astype(f32).T, att_dst2.astype(f32).T], axis=1)
    ei32 = edge_index.astype(jnp.int32)
    eint = jnp.stack([ei32[0].reshape(_E // _B, _B),
                      ei32[1].reshape(_E // _B, _B)], axis=1).reshape(-1)
    eint = jnp.concatenate([eint, jnp.zeros((_EINT_LEN - 2 * _E,), jnp.int32)])
    zeros = jnp.zeros((_NPAD, 128), f32)

    t1 = pl.pallas_call(
        _tc_stage_a,
        out_shape=jax.ShapeDtypeStruct((4 * _N, 128), f32),
    )(x.astype(f32), W1.astype(f32), aproj)

    acc1 = _make_sc_call(_sc_edges1)(t1, eint, zeros)

    t2 = pl.pallas_call(
        _tc_stage_b,
        out_shape=jax.ShapeDtypeStruct((2 * _N, 128), f32),
    )(acc1, t1[0:2 * _N], b1.astype(f32).reshape(1, _D1), W2.astype(f32), r_mat, a2)

    acc2 = _make_sc_call(_sc_edges2)(t2, eint, zeros)

    out = pl.pallas_call(
        _tc_stage_c,
        out_shape=jax.ShapeDtypeStruct((_N, _D_OUT), f32),
    )(acc2, t2[0:_N], b2.astype(f32).reshape(1, _D_OUT))
    return out
